# baseline (speedup 1.0000x reference)
import functools

import jax
import jax.numpy as jnp
from jax import lax
from jax.experimental import pallas as pl
from jax.experimental.pallas import tpu as pltpu
from jax.experimental.pallas import tpu_sc as plsc

_NC = 2
_NS = 16
_NW = _NC * _NS
_GCHUNK = 8
_SCHUNK = 32
_RING = 96
_INFLIGHT = 8
_GPS = _SCHUNK // _GCHUNK
_GSLOTS = _RING // _GCHUNK
_SSLOTS = _RING // _SCHUNK


@functools.cache
def _make_gather(B, D):
    b_per_w = B // _NW
    n_g = b_per_w // _GCHUNK
    n_s = b_per_w // _SCHUNK
    mesh = plsc.VectorSubcoreMesh(core_axis_name="c", subcore_axis_name="s")

    @functools.partial(
        pl.kernel,
        mesh=mesh,
        out_type=jax.ShapeDtypeStruct((B, D), jnp.float32),
        scratch_types=(
            [pltpu.VMEM((n_g, _GCHUNK), jnp.int32),
             pltpu.VMEM((_RING, D), jnp.float32)]
            + [pltpu.SemaphoreType.DMA for _ in range(_GSLOTS + _SSLOTS)]
        ),
    )
    def gather_kernel(idx_hbm, table_hbm, out_hbm, idx_v, ring, *sems):
        gsems = sems[:_GSLOTS]
        ssems = sems[_GSLOTS:]
        wid = lax.axis_index("s") * _NC + lax.axis_index("c")
        base = wid * b_per_w
        pltpu.sync_copy(idx_hbm.at[wid], idx_v)

        def gather(j):
            slot = j % _GSLOTS
            return pltpu.async_copy(
                table_hbm.at[idx_v.at[j]],
                ring.at[pl.ds(slot * _GCHUNK, _GCHUNK)],
                gsems[slot],
            )

        def scatter(g):
            slot = g % _SSLOTS
            return pltpu.async_copy(
                ring.at[pl.ds(slot * _SCHUNK, _SCHUNK)],
                out_hbm.at[pl.ds(base + g * _SCHUNK, _SCHUNK)],
                ssems[slot],
            )

        gathers = [None] * n_g
        scatters = [None] * n_s
        s_waited = [False] * n_s
        prime = min(_INFLIGHT, n_g)
        for j in range(prime):
            gathers[j] = gather(j)
        for j in range(n_g):
            gathers[j].wait()
            if j % _GPS == _GPS - 1:
                scatters[j // _GPS] = scatter(j // _GPS)
            nx = j + prime
            if nx < n_g:
                pg = (nx - _GSLOTS) // _GPS
                if nx >= _GSLOTS and not s_waited[pg]:
                    scatters[pg].wait()
                    s_waited[pg] = True
                gathers[nx] = gather(nx)
        for g in range(n_s):
            if not s_waited[g]:
                scatters[g].wait()

    return gather_kernel


def kernel(x, W):
    Bx, S = x.shape
    D = W.shape[1]
    idx = x.reshape(-1).astype(jnp.int32)
    B = idx.size
    idx3 = idx.reshape(_NW, B // _NW // _GCHUNK, _GCHUNK)
    out = _make_gather(B, D)(idx3, W)
    return out.reshape(Bx, S, D)
